# baseline (speedup 1.0000x reference)
import functools

import jax
import jax.numpy as jnp
from jax import lax
from jax.experimental import pallas as pl
from jax.experimental.pallas import tpu as pltpu
from jax.experimental.pallas import tpu_sc as plsc

N_SOFT = 10
LANES = 16


@functools.lru_cache(maxsize=None)
def _build(B, L, V, D):
    info = plsc.get_sparse_core_info()
    num_cores = info.num_cores
    num_workers = num_cores * info.num_subcores
    n_rows = B * L
    assert n_rows % num_workers == 0
    rows_per_w = n_rows // num_workers
    chunk = 32
    nbuf = 4
    assert rows_per_w % chunk == 0
    n_chunks = rows_per_w // chunk

    mesh = plsc.VectorSubcoreMesh(core_axis_name="c", subcore_axis_name="s")

    @functools.partial(
        pl.kernel,
        mesh=mesh,
        out_type=jax.ShapeDtypeStruct((n_rows, D), jnp.float32),
        scratch_types=[
            pltpu.VMEM((rows_per_w,), jnp.int32),
            pltpu.VMEM((N_SOFT, D), jnp.float32),
        ] + [pltpu.VMEM((chunk, D), jnp.float32)] * nbuf
          + [pltpu.SemaphoreType.DMA] * (2 * nbuf + 1),
    )
    def run(tokens_hbm, wte_hbm, sp_hbm, out_hbm, idx_v, sp_v, *rest):
        bufs = rest[:nbuf]
        gsems = rest[nbuf:2 * nbuf]
        osems = rest[2 * nbuf:3 * nbuf]
        spsem = rest[3 * nbuf]
        wid = lax.axis_index("c") * info.num_subcores + lax.axis_index("s")
        base = wid * rows_per_w

        sp_copy = pltpu.async_copy(sp_hbm, sp_v, spsem)
        pltpu.sync_copy(tokens_hbm.at[pl.ds(base, rows_per_w)], idx_v)

        def start_gather(c):
            return pltpu.async_copy(
                wte_hbm.at[idx_v.at[pl.ds(c * chunk, chunk)]],
                bufs[c % nbuf], gsems[c % nbuf])

        gathers = [None] * n_chunks
        outs = [None] * n_chunks
        for c in range(min(nbuf - 1, n_chunks)):
            gathers[c] = start_gather(c)
        for c in range(n_chunks):
            buf = bufs[c % nbuf]
            gathers[c].wait()
            if c == 0:
                sp_copy.wait()

                @pl.when(base % L == 0)
                def _():
                    def body(j, carry):
                        col = j * LANES
                        for l in range(N_SOFT):
                            buf[l, pl.ds(col, LANES)] = (
                                sp_v[l, pl.ds(col, LANES)])
                        return carry

                    lax.fori_loop(0, D // LANES, body, 0)
            outs[c] = pltpu.async_copy(
                buf, out_hbm.at[pl.ds(base + c * chunk, chunk)], osems[c % nbuf])
            nxt = c + nbuf - 1
            if nxt < n_chunks:
                if c >= 1:
                    outs[c - 1].wait()
                gathers[nxt] = start_gather(nxt)
        for c in range(max(0, n_chunks - nbuf + 1), n_chunks):
            if c >= 1:
                outs[c - 1].wait()
        outs[n_chunks - 1].wait()

    return run


def kernel(tokens, wte, soft_prompt):
    B, L = tokens.shape
    V, D = wte.shape
    run = _build(B, L, V, D)
    flat = run(tokens.reshape(-1).astype(jnp.int32), wte, soft_prompt)
    return flat.reshape(B, L, D)


# speedup vs baseline: 1.0109x; 1.0109x over previous
import functools

import jax
import jax.numpy as jnp
from jax import lax
from jax.experimental import pallas as pl
from jax.experimental.pallas import tpu as pltpu
from jax.experimental.pallas import tpu_sc as plsc

N_SOFT = 10
LANES = 16


@functools.lru_cache(maxsize=None)
def _build(B, L, V, D):
    info = plsc.get_sparse_core_info()
    num_cores = info.num_cores
    num_workers = num_cores * info.num_subcores
    n_rows = B * L
    assert n_rows % num_workers == 0
    rows_per_w = n_rows // num_workers
    chunk = 16
    nbuf = 8
    assert rows_per_w % chunk == 0
    n_chunks = rows_per_w // chunk

    mesh = plsc.VectorSubcoreMesh(core_axis_name="c", subcore_axis_name="s")

    @functools.partial(
        pl.kernel,
        mesh=mesh,
        out_type=jax.ShapeDtypeStruct((n_rows, D), jnp.float32),
        scratch_types=[
            pltpu.VMEM((rows_per_w,), jnp.int32),
            pltpu.VMEM((N_SOFT, D), jnp.float32),
        ] + [pltpu.VMEM((chunk, D), jnp.float32)] * nbuf
          + [pltpu.SemaphoreType.DMA] * (2 * nbuf),
    )
    def run(tokens_hbm, wte_hbm, sp_hbm, out_hbm, idx_v, sp_v, *rest):
        bufs = rest[:nbuf]
        gsems = rest[nbuf:2 * nbuf]
        osems = rest[2 * nbuf:]
        wid = lax.axis_index("s") * num_cores + lax.axis_index("c")
        base = wid * rows_per_w

        pltpu.sync_copy(tokens_hbm.at[pl.ds(base, rows_per_w)], idx_v)

        def start_gather(c):
            return pltpu.async_copy(
                wte_hbm.at[idx_v.at[pl.ds(c * chunk, chunk)]],
                bufs[c % nbuf], gsems[c % nbuf])

        gathers = [None] * n_chunks
        outs = [None] * n_chunks
        for c in range(min(nbuf - 1, n_chunks)):
            gathers[c] = start_gather(c)
        for c in range(n_chunks):
            buf = bufs[c % nbuf]
            gathers[c].wait()
            if c == 0:
                @pl.when(base % L == 0)
                def _():
                    pltpu.sync_copy(sp_hbm, sp_v)

                    def body(j, carry):
                        col = j * LANES
                        for l in range(N_SOFT):
                            buf[l, pl.ds(col, LANES)] = (
                                sp_v[l, pl.ds(col, LANES)])
                        return carry

                    lax.fori_loop(0, D // LANES, body, 0)
            outs[c] = pltpu.async_copy(
                buf, out_hbm.at[pl.ds(base + c * chunk, chunk)], osems[c % nbuf])
            nxt = c + nbuf - 1
            if nxt < n_chunks:
                if c >= 1:
                    outs[c - 1].wait()
                gathers[nxt] = start_gather(nxt)
        for c in range(max(0, n_chunks - nbuf + 1), n_chunks):
            if c >= 1:
                outs[c - 1].wait()
        outs[n_chunks - 1].wait()

    return run


def kernel(tokens, wte, soft_prompt):
    B, L = tokens.shape
    V, D = wte.shape
    run = _build(B, L, V, D)
    flat = run(tokens.reshape(-1).astype(jnp.int32), wte, soft_prompt)
    return flat.reshape(B, L, D)


# speedup vs baseline: 1.0137x; 1.0028x over previous
import functools

import jax
import jax.numpy as jnp
from jax import lax
from jax.experimental import pallas as pl
from jax.experimental.pallas import tpu as pltpu
from jax.experimental.pallas import tpu_sc as plsc

N_SOFT = 10
LANES = 16


@functools.lru_cache(maxsize=None)
def _build(B, L, V, D):
    info = plsc.get_sparse_core_info()
    num_cores = info.num_cores
    num_workers = num_cores * info.num_subcores
    n_rows = B * L
    assert n_rows % num_workers == 0
    rows_per_w = n_rows // num_workers
    chunk = 32
    nbuf = 4
    assert rows_per_w % chunk == 0
    n_chunks = rows_per_w // chunk

    mesh = plsc.VectorSubcoreMesh(core_axis_name="c", subcore_axis_name="s")

    @functools.partial(
        pl.kernel,
        mesh=mesh,
        out_type=jax.ShapeDtypeStruct((n_rows, D), jnp.float32),
        scratch_types=[
            pltpu.VMEM((rows_per_w,), jnp.int32),
            pltpu.VMEM((N_SOFT, D), jnp.float32),
        ] + [pltpu.VMEM((chunk, D), jnp.float32)] * nbuf
          + [pltpu.SemaphoreType.DMA] * (2 * nbuf),
    )
    def run(tokens_hbm, wte_hbm, sp_hbm, out_hbm, idx_v, sp_v, *rest):
        bufs = rest[:nbuf]
        gsems = rest[nbuf:2 * nbuf]
        osems = rest[2 * nbuf:]
        wid = lax.axis_index("s") * num_cores + lax.axis_index("c")
        base = wid * rows_per_w

        def start_gather(c):
            return pltpu.async_copy(
                wte_hbm.at[idx_v.at[pl.ds(c * chunk, chunk)]],
                bufs[c % nbuf], gsems[c % nbuf])

        gathers = [None] * n_chunks
        outs = [None] * n_chunks
        pltpu.sync_copy(tokens_hbm.at[pl.ds(base, chunk)],
                        idx_v.at[pl.ds(0, chunk)])
        gathers[0] = start_gather(0)
        pltpu.sync_copy(
            tokens_hbm.at[pl.ds(base + chunk, rows_per_w - chunk)],
            idx_v.at[pl.ds(chunk, rows_per_w - chunk)])
        for c in range(1, min(nbuf - 1, n_chunks)):
            gathers[c] = start_gather(c)
        for c in range(n_chunks):
            buf = bufs[c % nbuf]
            gathers[c].wait()
            if c == 0:
                @pl.when(base % L == 0)
                def _():
                    pltpu.sync_copy(sp_hbm, sp_v)

                    def body(j, carry):
                        col = j * LANES
                        for l in range(N_SOFT):
                            buf[l, pl.ds(col, LANES)] = (
                                sp_v[l, pl.ds(col, LANES)])
                        return carry

                    lax.fori_loop(0, D // LANES, body, 0)
            outs[c] = pltpu.async_copy(
                buf, out_hbm.at[pl.ds(base + c * chunk, chunk)], osems[c % nbuf])
            nxt = c + nbuf - 1
            if nxt < n_chunks:
                if c >= 1:
                    outs[c - 1].wait()
                gathers[nxt] = start_gather(nxt)
        for c in range(max(0, n_chunks - nbuf + 1), n_chunks):
            if c >= 1:
                outs[c - 1].wait()
        outs[n_chunks - 1].wait()

    return run


def kernel(tokens, wte, soft_prompt):
    B, L = tokens.shape
    V, D = wte.shape
    run = _build(B, L, V, D)
    flat = run(tokens.reshape(-1).astype(jnp.int32), wte, soft_prompt)
    return flat.reshape(B, L, D)


# speedup vs baseline: 1.0233x; 1.0095x over previous
import functools

import jax
import jax.numpy as jnp
from jax import lax
from jax.experimental import pallas as pl
from jax.experimental.pallas import tpu as pltpu
from jax.experimental.pallas import tpu_sc as plsc

N_SOFT = 10
LANES = 16


@functools.lru_cache(maxsize=None)
def _build(B, L, V, D):
    info = plsc.get_sparse_core_info()
    num_cores = info.num_cores
    num_workers = num_cores * info.num_subcores
    n_rows = B * L
    assert n_rows % num_workers == 0
    rows_per_w = n_rows // num_workers
    chunk = 32
    nbuf = 4
    assert rows_per_w % chunk == 0
    n_chunks = rows_per_w // chunk

    mesh = plsc.VectorSubcoreMesh(core_axis_name="c", subcore_axis_name="s")

    w_per_row = L // rows_per_w

    @functools.partial(
        pl.kernel,
        mesh=mesh,
        out_type=jax.ShapeDtypeStruct((n_rows, D), jnp.float32),
        scratch_types=[
            pltpu.VMEM((B, rows_per_w), jnp.int32),
            pltpu.VMEM((rows_per_w,), jnp.int32),
            pltpu.VMEM((N_SOFT, D), jnp.float32),
        ] + [pltpu.VMEM((chunk, D), jnp.float32)] * nbuf
          + [pltpu.SemaphoreType.DMA] * (2 * nbuf),
    )
    def run(tokens_hbm, wte_hbm, sp_hbm, out_hbm, tok_v, idx_v, sp_v, *rest):
        bufs = rest[:nbuf]
        gsems = rest[nbuf:2 * nbuf]
        osems = rest[2 * nbuf:]
        wid = lax.axis_index("s") * num_cores + lax.axis_index("c")
        base = wid * rows_per_w

        b_row = wid // w_per_row
        col0 = (wid % w_per_row) * rows_per_w
        pltpu.sync_copy(tokens_hbm.at[:, pl.ds(col0, rows_per_w)], tok_v)

        def idx_body(j, carry):
            col = j * LANES
            idx_v[pl.ds(col, LANES)] = tok_v[b_row, pl.ds(col, LANES)]
            return carry

        lax.fori_loop(0, rows_per_w // LANES, idx_body, 0)

        def start_gather(c):
            return pltpu.async_copy(
                wte_hbm.at[idx_v.at[pl.ds(c * chunk, chunk)]],
                bufs[c % nbuf], gsems[c % nbuf])

        gathers = [None] * n_chunks
        outs = [None] * n_chunks
        for c in range(min(nbuf - 1, n_chunks)):
            gathers[c] = start_gather(c)
        for c in range(n_chunks):
            buf = bufs[c % nbuf]
            gathers[c].wait()
            if c == 0:
                @pl.when(base % L == 0)
                def _():
                    pltpu.sync_copy(sp_hbm, sp_v)

                    def body(j, carry):
                        col = j * LANES
                        for l in range(N_SOFT):
                            buf[l, pl.ds(col, LANES)] = (
                                sp_v[l, pl.ds(col, LANES)])
                        return carry

                    lax.fori_loop(0, D // LANES, body, 0)
            outs[c] = pltpu.async_copy(
                buf, out_hbm.at[pl.ds(base + c * chunk, chunk)], osems[c % nbuf])
            nxt = c + nbuf - 1
            if nxt < n_chunks:
                if c >= 1:
                    outs[c - 1].wait()
                gathers[nxt] = start_gather(nxt)
        for c in range(max(0, n_chunks - nbuf + 1), n_chunks):
            if c >= 1:
                outs[c - 1].wait()
        outs[n_chunks - 1].wait()

    return run


def kernel(tokens, wte, soft_prompt):
    B, L = tokens.shape
    V, D = wte.shape
    run = _build(B, L, V, D)
    flat = run(tokens.astype(jnp.int32), wte, soft_prompt)
    return flat.reshape(B, L, D)


# speedup vs baseline: 1.0312x; 1.0078x over previous
import functools

import jax
import jax.numpy as jnp
from jax import lax
from jax.experimental import pallas as pl
from jax.experimental.pallas import tpu as pltpu
from jax.experimental.pallas import tpu_sc as plsc

N_SOFT = 10
LANES = 16


@functools.lru_cache(maxsize=None)
def _build(B, L, V, D):
    info = plsc.get_sparse_core_info()
    num_cores = info.num_cores
    num_workers = num_cores * info.num_subcores
    n_rows = B * L
    assert n_rows % num_workers == 0
    rows_per_w = n_rows // num_workers
    chunk = 32
    nbuf = 4
    assert rows_per_w % chunk == 0
    n_chunks = rows_per_w // chunk

    mesh = plsc.VectorSubcoreMesh(core_axis_name="c", subcore_axis_name="s")

    @functools.partial(
        pl.kernel,
        mesh=mesh,
        out_type=jax.ShapeDtypeStruct((n_rows, D), jnp.float32),
        scratch_types=[
            pltpu.VMEM((rows_per_w,), jnp.int32),
            pltpu.VMEM((N_SOFT, D), jnp.float32),
        ] + [pltpu.VMEM((chunk, D), jnp.float32)] * nbuf
          + [pltpu.SemaphoreType.DMA] * (2 * nbuf),
    )
    def run(tokens_hbm, wte_hbm, sp_hbm, out_hbm, idx_v, sp_v, *rest):
        bufs = rest[:nbuf]
        gsems = rest[nbuf:2 * nbuf]
        osems = rest[2 * nbuf:]
        wid = lax.axis_index("s") * num_cores + lax.axis_index("c")
        base = wid * rows_per_w

        pltpu.sync_copy(tokens_hbm.at[pl.ds(base, rows_per_w)], idx_v)

        def start_gather(c):
            return pltpu.async_copy(
                wte_hbm.at[idx_v.at[pl.ds(c * chunk, chunk)]],
                bufs[c % nbuf], gsems[c % nbuf])

        gathers = [None] * n_chunks
        outs = [None] * n_chunks
        for c in range(min(nbuf - 1, n_chunks)):
            gathers[c] = start_gather(c)
        for c in range(n_chunks):
            buf = bufs[c % nbuf]
            gathers[c].wait()
            if c == 0:
                @pl.when(base % L == 0)
                def _():
                    pltpu.sync_copy(sp_hbm, sp_v)

                    def body(j, carry):
                        col = j * LANES
                        for l in range(N_SOFT):
                            buf[l, pl.ds(col, LANES)] = (
                                sp_v[l, pl.ds(col, LANES)])
                        return carry

                    lax.fori_loop(0, D // LANES, body, 0)
            outs[c] = pltpu.async_copy(
                buf, out_hbm.at[pl.ds(base + c * chunk, chunk)], osems[c % nbuf])
            nxt = c + nbuf - 1
            if nxt < n_chunks:
                if c >= 1:
                    outs[c - 1].wait()
                gathers[nxt] = start_gather(nxt)
        for c in range(max(0, n_chunks - nbuf + 1), n_chunks):
            if c >= 1:
                outs[c - 1].wait()
        outs[n_chunks - 1].wait()

    return run


def kernel(tokens, wte, soft_prompt):
    B, L = tokens.shape
    V, D = wte.shape
    run = _build(B, L, V, D)
    flat = run(tokens.reshape(-1).astype(jnp.int32), wte, soft_prompt)
    return flat.reshape(B, L, D)


# speedup vs baseline: 1.0349x; 1.0035x over previous
import functools

import jax
import jax.numpy as jnp
from jax import lax
from jax.experimental import pallas as pl
from jax.experimental.pallas import tpu as pltpu
from jax.experimental.pallas import tpu_sc as plsc

N_SOFT = 10
LANES = 16


@functools.lru_cache(maxsize=None)
def _build(B, L, V, D):
    info = plsc.get_sparse_core_info()
    num_cores = info.num_cores
    num_workers = num_cores * info.num_subcores
    n_rows = B * L
    assert n_rows % num_workers == 0
    rows_per_w = n_rows // num_workers
    chunk = 32
    nbuf = 4
    assert rows_per_w % chunk == 0
    assert L % rows_per_w == 0
    assert N_SOFT <= chunk
    n_chunks = rows_per_w // chunk

    mesh = plsc.VectorSubcoreMesh(core_axis_name="c", subcore_axis_name="s")

    @functools.partial(
        pl.kernel,
        mesh=mesh,
        out_type=jax.ShapeDtypeStruct((n_rows, D), jnp.float32),
        scratch_types=[
            pltpu.VMEM((rows_per_w,), jnp.int32),
            pltpu.VMEM((N_SOFT, D), jnp.float32),
        ] + [pltpu.VMEM((chunk, D), jnp.float32)] * nbuf
          + [pltpu.SemaphoreType.DMA] * (2 * nbuf),
    )
    def run(tokens_hbm, wte_hbm, sp_hbm, out_hbm, idx_v, sp_v, *rest):
        bufs = rest[:nbuf]
        gsems = rest[nbuf:2 * nbuf]
        osems = rest[2 * nbuf:]
        wid = lax.axis_index("s") * num_cores + lax.axis_index("c")
        base = wid * rows_per_w

        pltpu.sync_copy(tokens_hbm.at[pl.ds(base, rows_per_w)], idx_v)

        def start_gather(c):
            return pltpu.async_copy(
                wte_hbm.at[idx_v.at[pl.ds(c * chunk, chunk)]],
                bufs[c % nbuf], gsems[c % nbuf])

        gathers = [None] * n_chunks
        outs = [None] * n_chunks
        for c in range(min(nbuf - 1, n_chunks)):
            gathers[c] = start_gather(c)
        for c in range(n_chunks):
            buf = bufs[c % nbuf]
            gathers[c].wait()
            if c == 0:
                @pl.when(base % L == 0)
                def _():
                    pltpu.sync_copy(sp_hbm, sp_v)

                    def body(j, carry):
                        col = j * LANES
                        for l in range(N_SOFT):
                            buf[l, pl.ds(col, LANES)] = (
                                sp_v[l, pl.ds(col, LANES)])
                        return carry

                    lax.fori_loop(0, D // LANES, body, 0)
            outs[c] = pltpu.async_copy(
                buf, out_hbm.at[pl.ds(base + c * chunk, chunk)], osems[c % nbuf])
            nxt = c + nbuf - 1
            if nxt < n_chunks:
                if c >= 1:
                    outs[c - 1].wait()
                gathers[nxt] = start_gather(nxt)
        for c in range(max(0, n_chunks - nbuf + 1), n_chunks):
            if c >= 1:
                outs[c - 1].wait()
        outs[n_chunks - 1].wait()

    return run


def kernel(tokens, wte, soft_prompt):
    B, L = tokens.shape
    V, D = wte.shape
    run = _build(B, L, V, D)
    flat = run(tokens.reshape(-1).astype(jnp.int32), wte, soft_prompt)
    return flat.reshape(B, L, D)
